# baseline (speedup 1.0000x reference)
import functools

import jax
import jax.numpy as jnp
from jax import lax
from jax.experimental import pallas as pl
from jax.experimental.pallas import tpu as pltpu
from jax.experimental.pallas import tpu_sc as plsc

D_MODEL = 64
LANE = 128
NUM_CORES = 2
NUM_SUBCORES = 16
NUM_WORKERS = NUM_CORES * NUM_SUBCORES
CHUNK_ROWS = 8


@functools.lru_cache(maxsize=None)
def _build_gather(num_rows: int, vocab: int, d_model: int):
    assert num_rows % NUM_WORKERS == 0
    rows_per_w = num_rows // NUM_WORKERS
    assert rows_per_w % CHUNK_ROWS == 0
    n_chunks = rows_per_w // CHUNK_ROWS

    mesh = plsc.VectorSubcoreMesh(core_axis_name="c", subcore_axis_name="s")

    @functools.partial(
        pl.kernel,
        mesh=mesh,
        out_type=jax.ShapeDtypeStruct((num_rows, LANE, d_model), jnp.float32),
        scratch_types=[
            pltpu.VMEM((CHUNK_ROWS, LANE), jnp.int32),
            pltpu.VMEM((CHUNK_ROWS, LANE, d_model), jnp.float32),
            pltpu.SemaphoreType.DMA,
        ],
        compiler_params=pltpu.CompilerParams(use_tc_tiling_on_sc=False),
    )
    def gather_kernel(table_hbm, idx_hbm, out_hbm, idx_v, rows_v, sem):
        wid = lax.axis_index("s") * NUM_CORES + lax.axis_index("c")
        row0 = wid * rows_per_w

        def step(i, carry):
            base = row0 + i * CHUNK_ROWS
            pltpu.sync_copy(idx_hbm.at[pl.ds(base, CHUNK_ROWS)], idx_v)
            copies = [
                pltpu.async_copy(table_hbm.at[idx_v.at[j]], rows_v.at[j], sem)
                for j in range(CHUNK_ROWS)
            ]
            for cp in copies:
                cp.wait()
            pltpu.sync_copy(rows_v, out_hbm.at[pl.ds(base, CHUNK_ROWS)])
            return carry

        lax.fori_loop(0, n_chunks, step, 0)

    return gather_kernel


def kernel(x, table):
    batch, hist = x.shape
    vocab, d_model = table.shape
    n = batch * hist
    assert n % LANE == 0
    num_rows = n // LANE
    idx = x.reshape(num_rows, LANE).astype(jnp.int32)
    out = _build_gather(num_rows, vocab, d_model)(table, idx)
    return out.reshape(batch, hist, d_model)


# speedup vs baseline: 1.0310x; 1.0310x over previous
import functools

import jax
import jax.numpy as jnp
from jax import lax
from jax.experimental import pallas as pl
from jax.experimental.pallas import tpu as pltpu
from jax.experimental.pallas import tpu_sc as plsc

D_MODEL = 64
LANE = 128
NUM_CORES = 2
NUM_SUBCORES = 16
NUM_WORKERS = NUM_CORES * NUM_SUBCORES
CHUNK_ROWS = 2
NBUF = 5
GATHER_LAT = 3


@functools.lru_cache(maxsize=None)
def _build_gather(num_rows: int, vocab: int, d_model: int):
    assert num_rows % NUM_WORKERS == 0
    rows_per_w = num_rows // NUM_WORKERS
    assert rows_per_w % (CHUNK_ROWS * NBUF) == 0
    n_chunks = rows_per_w // CHUNK_ROWS
    n_groups = n_chunks // NBUF

    mesh = plsc.VectorSubcoreMesh(core_axis_name="c", subcore_axis_name="s")

    @functools.partial(
        pl.kernel,
        mesh=mesh,
        out_type=jax.ShapeDtypeStruct((num_rows, LANE, d_model), jnp.float32),
        scratch_types=[
            pltpu.VMEM((NBUF * CHUNK_ROWS, LANE), jnp.int32),
            pltpu.VMEM((NBUF * CHUNK_ROWS, LANE, d_model), jnp.float32),
        ]
        + [pltpu.SemaphoreType.DMA] * (2 * NBUF),
        compiler_params=pltpu.CompilerParams(use_tc_tiling_on_sc=False),
    )
    def gather_kernel(table_hbm, idx_hbm, out_hbm, idx_v, rows_v, *sems):
        sem_g = sems[:NBUF]
        sem_s = sems[NBUF:]
        wid = lax.axis_index("s") * NUM_CORES + lax.axis_index("c")
        row0 = wid * rows_per_w

        def load_idx(i, b):
            pltpu.sync_copy(
                idx_hbm.at[pl.ds(row0 + i * CHUNK_ROWS, CHUNK_ROWS)],
                idx_v.at[pl.ds(b * CHUNK_ROWS, CHUNK_ROWS)],
            )

        def start_gather(b):
            for j in range(CHUNK_ROWS):
                pltpu.async_copy(
                    table_hbm.at[idx_v.at[b * CHUNK_ROWS + j]],
                    rows_v.at[b * CHUNK_ROWS + j],
                    sem_g[b],
                )

        def wait_gather(b):
            for j in range(CHUNK_ROWS):
                pltpu.make_async_copy(
                    table_hbm.at[idx_v.at[b * CHUNK_ROWS + j]],
                    rows_v.at[b * CHUNK_ROWS + j],
                    sem_g[b],
                ).wait()

        def start_store(i, b):
            pltpu.async_copy(
                rows_v.at[pl.ds(b * CHUNK_ROWS, CHUNK_ROWS)],
                out_hbm.at[pl.ds(row0 + i * CHUNK_ROWS, CHUNK_ROWS)],
                sem_s[b],
            )

        def wait_store(i, b):
            pltpu.make_async_copy(
                rows_v.at[pl.ds(b * CHUNK_ROWS, CHUNK_ROWS)],
                out_hbm.at[pl.ds(row0 + i * CHUNK_ROWS, CHUNK_ROWS)],
                sem_s[b],
            ).wait()

        for i in range(NBUF):
            load_idx(i, i)
            start_gather(i)
            if i >= GATHER_LAT:
                c = i - GATHER_LAT
                wait_gather(c % NBUF)
                start_store(c, c % NBUF)

        def group(g, carry):
            for b in range(NBUF):
                i = g * NBUF + b
                wait_store(i - NBUF, b)
                load_idx(i, b)
                start_gather(b)
                bd = (b - GATHER_LAT) % NBUF
                wait_gather(bd)
                start_store(i - GATHER_LAT, bd)
            return carry

        lax.fori_loop(1, n_groups, group, 0)

        for c in range(n_chunks - GATHER_LAT, n_chunks):
            wait_gather(c % NBUF)
            start_store(c, c % NBUF)
        for c in range(n_chunks - NBUF, n_chunks):
            wait_store(c, c % NBUF)

    return gather_kernel


def kernel(x, table):
    batch, hist = x.shape
    vocab, d_model = table.shape
    n = batch * hist
    assert n % LANE == 0
    num_rows = n // LANE
    idx = x.reshape(num_rows, LANE).astype(jnp.int32)
    out = _build_gather(num_rows, vocab, d_model)(table, idx)
    return out.reshape(batch, hist, d_model)
